# baseline (speedup 1.0000x reference)
import functools

import jax
import jax.numpy as jnp
from jax import lax
from jax.experimental import pallas as pl
from jax.experimental.pallas import tpu as pltpu
from jax.experimental.pallas import tpu_sc as plsc

B = 4096
CTX = 20
D = 128
V = 100000

NC = 2
NS = 16
NW = NC * NS
CH = 4
IPC = CH * CTX

HALVES = 2
BH = B // HALVES


def _gather_mean_sc(idx_flat, emb, bsz):
  bpw = bsz // NW
  nchunk = bpw // CH
  mesh = plsc.VectorSubcoreMesh(core_axis_name="c", subcore_axis_name="s")

  @functools.partial(
      pl.kernel,
      mesh=mesh,
      out_type=jax.ShapeDtypeStruct((bsz, D), jnp.float32),
      scratch_types=[
          pltpu.VMEM((bpw * CTX,), jnp.int32),
          pltpu.VMEM((2, IPC, D), jnp.float32),
          pltpu.VMEM((bpw, D), jnp.float32),
          pltpu.SemaphoreType.DMA,
          pltpu.SemaphoreType.DMA,
      ],
      compiler_params=pltpu.CompilerParams(use_tc_tiling_on_sc=True),
  )
  def k(idx_hbm, emb_hbm, ctx_hbm, idx_v, rows_v, acc_v, sem_a, sem_b):
    wid = lax.axis_index("s") * NC + lax.axis_index("c")
    base = wid * bpw
    pltpu.sync_copy(idx_hbm.at[pl.ds(base * CTX, bpw * CTX)], idx_v)
    pltpu.async_copy(
        emb_hbm.at[idx_v.at[pl.ds(0, IPC)]], rows_v.at[0], sem_a)

    def accumulate(buf, k_chunk):
      def per_row(r, _):
        src = r * CTX
        for dd in range(D // 16):
          sl = pl.ds(dd * 16, 16)
          a = rows_v[buf, src, sl]
          for ci in range(1, CTX):
            a = a + rows_v[buf, src + ci, sl]
          acc_v[k_chunk * CH + r, sl] = a * (1.0 / CTX)
        return 0
      lax.fori_loop(0, CH, per_row, 0)

    def body(i, _):
      k0 = 2 * i
      k1 = k0 + 1
      pltpu.async_copy(
          emb_hbm.at[idx_v.at[pl.ds(k1 * IPC, IPC)]], rows_v.at[1], sem_b)
      pltpu.make_async_copy(
          emb_hbm.at[idx_v.at[pl.ds(k0 * IPC, IPC)]], rows_v.at[0],
          sem_a).wait()
      accumulate(0, k0)

      @pl.when(k0 + 2 < nchunk)
      def _():
        pltpu.async_copy(
            emb_hbm.at[idx_v.at[pl.ds((k0 + 2) * IPC, IPC)]], rows_v.at[0],
            sem_a)

      pltpu.make_async_copy(
          emb_hbm.at[idx_v.at[pl.ds(k1 * IPC, IPC)]], rows_v.at[1],
          sem_b).wait()
      accumulate(1, k1)
      return 0

    lax.fori_loop(0, nchunk // 2, body, 0)
    pltpu.sync_copy(acc_v, ctx_hbm.at[pl.ds(base, bpw)])

  return k(idx_flat, emb)


BN = 1792
GRID_N = pl.cdiv(V, BN)

_TC_PARAMS = pltpu.CompilerParams(
    dimension_semantics=("parallel",),
    vmem_limit_bytes=63 * 1024 * 1024)


def _bias_col(b_ref):
  return jnp.transpose(b_ref[...], (1, 0))[:, 0:1]


def _proj_body(w_ref, ctx_ref, b_ref, out_ref):
  c = ctx_ref[...].astype(jnp.bfloat16)
  w = w_ref[...].astype(jnp.bfloat16)
  acc = lax.dot_general(w, c, (((1,), (1,)), ((), ())),
                        preferred_element_type=jnp.float32)
  out_ref[...] = acc + _bias_col(b_ref)


def _proj_body_alias(w_ref, ctx_ref, b_ref, prev_ref, out_ref):
  del prev_ref
  _proj_body(w_ref, ctx_ref, b_ref, out_ref)


def _project_half(W, ctx_h, b8, half, prev=None):
  in_specs = [
      pl.BlockSpec((BN, D), lambda n: (n, 0)),
      pl.BlockSpec((BH, D), lambda n: (0, 0)),
      pl.BlockSpec((8, BN), lambda n: (0, n)),
  ]
  args = [W, ctx_h, b8]
  body = _proj_body
  aliases = {}
  if prev is not None:
    in_specs.append(pl.BlockSpec((8, 128), lambda n: (0, 0)))
    args.append(prev)
    body = _proj_body_alias
    aliases = {3: 0}
  return pl.pallas_call(
      body,
      grid=(GRID_N,),
      in_specs=in_specs,
      out_specs=pl.BlockSpec((BN, BH), lambda n: (n, half)),
      out_shape=jax.ShapeDtypeStruct((V, B), jnp.float32),
      input_output_aliases=aliases,
      compiler_params=_TC_PARAMS,
  )(*args)


def kernel(X, emb, W, b):
  idx = X.astype(jnp.int32).reshape(-1)
  b8 = jnp.broadcast_to(b.reshape(1, V), (8, V))
  ctx0 = _gather_mean_sc(idx[: BH * CTX], emb, BH)
  ctx1 = _gather_mean_sc(idx[BH * CTX:], emb, BH)
  out = _project_half(W, ctx0, b8, 0)
  out = _project_half(W, ctx1, b8, 1, prev=out)
  return out.T
